# baseline (speedup 1.0000x reference)
import functools

import jax
import jax.numpy as jnp
from jax import lax
from jax.experimental import pallas as pl
from jax.experimental.pallas import tpu as pltpu
from jax.experimental.pallas import tpu_sc as plsc

N_ROWS = 16384
D_OBS = 128
N_PH = 8
D_OUT = D_OBS + N_PH
NC = 2
NS = 16
NW = NC * NS
ROWS_PER_W = N_ROWS // NW
CHUNK = 256
N_CHUNKS = ROWS_PER_W // CHUNK


def _sc_body(obs_hbm, ph_hbm, out_hbm, obs_v, ph_v, oh_v, sem):
    wid = lax.axis_index("s") * NC + lax.axis_index("c")
    base = wid * ROWS_PER_W

    lanes = lax.broadcasted_iota(jnp.int32, (16,), 0)
    sub = lanes & 7
    rows2 = lanes >> 3

    for k in range(N_CHUNKS):
        r0 = base + k * CHUNK
        cp = pltpu.make_async_copy(obs_hbm.at[pl.ds(r0, CHUNK)], obs_v, sem)
        cp.start()
        pltpu.sync_copy(ph_hbm.at[pl.ds(r0, CHUNK)], ph_v)

        def oh_body(j, carry):
            ph = plsc.load_gather(ph_v, [j * 2 + rows2])
            v = jnp.where(sub == ph, 1.0, 0.0).astype(jnp.float32)
            plsc.store_scatter(oh_v, [j * 2 + rows2, sub], v)
            return carry

        lax.fori_loop(0, CHUNK // 2, oh_body, 0)

        cp.wait()
        pltpu.sync_copy(obs_v, out_hbm.at[pl.ds(r0, CHUNK), pl.ds(0, D_OBS)])
        pltpu.sync_copy(oh_v, out_hbm.at[pl.ds(r0, CHUNK), pl.ds(D_OBS, N_PH)])


_mesh = plsc.VectorSubcoreMesh(core_axis_name="c", subcore_axis_name="s")

_sc_call = functools.partial(
    pl.kernel,
    mesh=_mesh,
    out_type=jax.ShapeDtypeStruct((N_ROWS, D_OUT), jnp.float32),
    scratch_types=[
        pltpu.VMEM((CHUNK, D_OBS), jnp.float32),
        pltpu.VMEM((CHUNK,), jnp.int32),
        pltpu.VMEM((CHUNK, N_PH), jnp.float32),
        pltpu.SemaphoreType.DMA,
    ],
    compiler_params=pltpu.CompilerParams(needs_layout_passes=False),
)(_sc_body)


def kernel(obs, phases):
    return _sc_call(obs, phases.astype(jnp.int32))


# speedup vs baseline: 1.0011x; 1.0011x over previous
import functools

import jax
import jax.numpy as jnp
from jax import lax
from jax.experimental import pallas as pl
from jax.experimental.pallas import tpu as pltpu
from jax.experimental.pallas import tpu_sc as plsc

N_ROWS = 16384
D_OBS = 128
N_PH = 8
D_OUT = D_OBS + N_PH
NC = 2
NS = 16
NW = NC * NS
ROWS_PER_W = N_ROWS // NW
CHUNK = 256
N_CHUNKS = ROWS_PER_W // CHUNK


def _sc_body(obs_hbm, ph_hbm, out_hbm, obs_v, ph_v, oh_v, sem):
    wid = lax.axis_index("s") * NC + lax.axis_index("c")
    base = wid * ROWS_PER_W

    lanes = lax.broadcasted_iota(jnp.int32, (16,), 0)
    sub = lanes & 7
    rows2 = lanes >> 3

    for k in range(N_CHUNKS):
        r0 = base + k * CHUNK
        cp = pltpu.make_async_copy(obs_hbm.at[pl.ds(r0, CHUNK)], obs_v, sem)
        cp.start()
        pltpu.sync_copy(ph_hbm.at[pl.ds(r0, CHUNK)], ph_v)

        def oh_body(j, carry):
            ph = plsc.load_gather(ph_v, [j * 2 + rows2])
            v = jnp.where(sub == ph, 1.0, 0.0).astype(jnp.float32)
            plsc.store_scatter(oh_v, [j * 2 + rows2, sub], v)
            return carry

        lax.fori_loop(0, CHUNK // 2, oh_body, 0)

        cp.wait()
        pltpu.sync_copy(obs_v, out_hbm.at[pl.ds(r0, CHUNK), pl.ds(0, D_OBS)])
        pltpu.sync_copy(oh_v, out_hbm.at[pl.ds(r0, CHUNK), pl.ds(D_OBS, N_PH)])


_mesh = plsc.VectorSubcoreMesh(core_axis_name="c", subcore_axis_name="s")

_sc_call = functools.partial(
    pl.kernel,
    mesh=_mesh,
    out_type=jax.ShapeDtypeStruct((N_ROWS, D_OUT), jnp.float32),
    scratch_types=[
        pltpu.VMEM((CHUNK, D_OBS), jnp.float32),
        pltpu.VMEM((CHUNK,), jnp.int32),
        pltpu.VMEM((CHUNK, N_PH), jnp.float32),
        pltpu.SemaphoreType.DMA,
    ],
    compiler_params=pltpu.CompilerParams(
        needs_layout_passes=False, use_tc_tiling_on_sc=True
    ),
)(_sc_body)


def kernel(obs, phases):
    return _sc_call(obs, phases.astype(jnp.int32))
